# baseline (speedup 1.0000x reference)
import functools

import jax
import jax.numpy as jnp
from jax import lax
from jax.experimental import pallas as pl
from jax.experimental.pallas import tpu as pltpu
from jax.experimental.pallas import tpu_sc as plsc


def _sc_gather(idx_flat, table, tc_tiling):
    B = idx_flat.shape[0]
    dim = table.shape[1]
    info = plsc.get_sparse_core_info()
    nw = info.num_cores * info.num_subcores
    b_per_w = B // nw
    assert b_per_w * nw == B and (b_per_w % 8) == 0

    mesh = plsc.VectorSubcoreMesh(core_axis_name="c", subcore_axis_name="s")

    @functools.partial(
        pl.kernel,
        mesh=mesh,
        compiler_params=pltpu.CompilerParams(use_tc_tiling_on_sc=tc_tiling),
        out_type=[
            jax.ShapeDtypeStruct((B, dim), jnp.float32),
        ],
        scratch_types=[
            pltpu.VMEM((b_per_w,), jnp.int32),
            pltpu.VMEM((b_per_w, dim), jnp.float32),
            pltpu.SemaphoreType.DMA,
        ],
    )
    def body(idx_hbm, t_hbm, out_hbm, idx_v, rows, sem):
        wid = lax.axis_index("s") * info.num_cores + lax.axis_index("c")
        base = wid * b_per_w
        pltpu.sync_copy(idx_hbm.at[pl.ds(base, b_per_w)], idx_v)
        pltpu.async_copy(t_hbm.at[idx_v], rows, sem).wait()
        pltpu.sync_copy(rows, out_hbm.at[pl.ds(base, b_per_w)])

    (out,) = body(idx_flat, table)
    return out


def kernel(idx, k_emb, v_emb):
    idx_flat = idx.reshape(-1).astype(jnp.int32)
    v_out = _sc_gather(idx_flat, v_emb, True)
    k_out = _sc_gather(idx_flat, k_emb, False)
    return (
        k_out.reshape(*idx.shape, k_emb.shape[1]),
        v_out.reshape(*idx.shape, v_emb.shape[1]),
    )
